# baseline (speedup 1.0000x reference)
import jax
import jax.numpy as jnp
from jax import lax
from jax.experimental import pallas as pl
from jax.experimental.pallas import tpu as pltpu
from jax.experimental.pallas import tpu_sc as plsc

N = 10000
C = 256
NR = 276480


def _sc_body(y, out, idx64, idx128, tmp, sem):
    sid = lax.axis_index("s")
    cid = lax.axis_index("c")
    iota16 = lax.iota(jnp.int32, 16)

    def fill_idx(buf, n, salt):
        def body(g, carry):
            v = (salt + g * 16 + iota16) * 9973 + sid * 613 + cid * 131
            buf[pl.ds(g * 16, 16)] = (v % (N * 27)) // 8 * 8
            return carry
        lax.fori_loop(0, n // 16, body, None)

    fill_idx(idx64, 64, 7)
    fill_idx(idx128, 128, 13)

    def _drain(rows, sem_):
        pltpu.make_async_copy(y.at[pl.ds(0, rows)],
                              tmp.at[pl.ds(0, rows)], sem_).wait()

    with jax.named_scope("gA_64"):
        def ga(i, carry):
            pltpu.async_copy(y.at[idx64], tmp.at[pl.ds(0, 64)], sem)
            _drain(64, sem)
            return carry
        lax.fori_loop(0, 135, ga, None)

    with jax.named_scope("gB_128"):
        def gb(i, carry):
            pltpu.async_copy(y.at[idx128], tmp, sem)
            _drain(128, sem)
            return carry
        lax.fori_loop(0, 68, gb, None)

    with jax.named_scope("gC_linear"):
        def gc(i, carry):
            pltpu.async_copy(y.at[pl.ds(i * 64, 64)],
                             tmp.at[pl.ds(0, 64)], sem)
            _drain(64, sem)
            return carry
        lax.fori_loop(0, 135, gc, None)

    with jax.named_scope("gD_64_pipe4"):
        def gd(i, carry):
            pltpu.async_copy(y.at[idx64], tmp.at[pl.ds(0, 64)], sem)
            pltpu.async_copy(y.at[idx64], tmp.at[pl.ds(0, 64)], sem)
            pltpu.async_copy(y.at[idx64], tmp.at[pl.ds(0, 64)], sem)
            pltpu.async_copy(y.at[idx64], tmp.at[pl.ds(0, 64)], sem)
            _drain(64, sem)
            _drain(64, sem)
            _drain(64, sem)
            _drain(64, sem)
            return carry
        lax.fori_loop(0, 34, gd, None)

    with jax.named_scope("gG_hotrow"):
        def fill_hot(g, carry):
            v = (g * 16 + iota16) * 9973 + sid * 613
            r = (v % (N * 27)) // 8 * 8
            hot = (v % 3) < 2
            idx64[pl.ds(g * 16, 16)] = jnp.where(hot, N * 27, r)
            return carry
        lax.fori_loop(0, 4, fill_hot, None)

        def gg(i, carry):
            pltpu.async_copy(y.at[idx64], tmp.at[pl.ds(0, 64)], sem)
            _drain(64, sem)
            return carry
        lax.fori_loop(0, 135, gg, None)

    with jax.named_scope("gE_acc"):
        def ge(i, carry):
            def body(r, c2):
                for c in range(C // 16):
                    sl = pl.ds(c * 16, 16)
                    plsc.addupdate(tmp.at[r, sl], tmp[r + 64, sl])
                return c2
            lax.fori_loop(0, 64, body, None)
            return carry
        lax.fori_loop(0, 135, ge, None)

    with jax.named_scope("gF_both"):
        def gf(i, carry):
            pltpu.async_copy(y.at[idx64], tmp.at[pl.ds(64, 64)], sem)
            _drain(64, sem)

            def body(r, c2):
                for c in range(C // 16):
                    sl = pl.ds(c * 16, 16)
                    plsc.addupdate(tmp.at[r, sl], tmp[r + 64, sl])
                return c2
            lax.fori_loop(0, 64, body, None)
            return carry
        lax.fori_loop(0, 135, gf, None)

    wid = cid * 16 + sid
    pltpu.sync_copy(tmp.at[pl.ds(0, 64)], out.at[pl.ds(wid * 64, 64)])


def kernel(features, inp_positions, W, voxel_size=1.0):
    y = jnp.zeros((NR, C), jnp.float32)
    mesh = plsc.VectorSubcoreMesh(core_axis_name="c", subcore_axis_name="s")
    out = pl.kernel(
        _sc_body,
        out_type=jax.ShapeDtypeStruct((2048, C), jnp.float32),
        mesh=mesh,
        scratch_types=[
            pltpu.VMEM((64,), jnp.int32),
            pltpu.VMEM((128,), jnp.int32),
            pltpu.VMEM((128, C), jnp.float32),
            pltpu.SemaphoreType.DMA,
        ],
        compiler_params=pltpu.CompilerParams(needs_layout_passes=False),
    )(y)
    return jnp.zeros((N, C), jnp.float32) + jnp.sum(out) * 0.0


# speedup vs baseline: 23.5894x; 23.5894x over previous
import jax
import jax.numpy as jnp
from jax import lax
from jax.experimental import pallas as pl
from jax.experimental.pallas import tpu as pltpu
from jax.experimental.pallas import tpu_sc as plsc

N = 10000
G = 32
GP = G + 2
C = 256
NOFF = 27
NP = 10240
TAB = 41984
PER_SC = NP // 16
ROWS = NP // 32
CH = 64
NCH = ROWS // CH
INVALID = N * NOFF
INIT_W = TAB // 16
SC_ROWS = 16 * ROWS
DOFF = [dx * GP * GP + dy * GP + dz
        for dx in (-1, 0, 1) for dy in (-1, 0, 1) for dz in (-1, 0, 1)]


def _mm_body(f_ref, w_ref, y_ref):
    y_ref[...] = jnp.dot(f_ref[...], w_ref[...],
                         preferred_element_type=jnp.float32)


def _big_matmul(f_pad, w_cat):
    bm, bn = 2048, 768
    grid = (NP // bm, (NOFF * C) // bn)
    return pl.pallas_call(
        _mm_body,
        grid=grid,
        in_specs=[
            pl.BlockSpec((bm, C), lambda i, j: (i, 0)),
            pl.BlockSpec((C, bn), lambda i, j: (0, j)),
        ],
        out_specs=pl.BlockSpec((bm, bn), lambda i, j: (i, j)),
        out_shape=jax.ShapeDtypeStruct((NP, NOFF * C), jnp.float32),
    )(f_pad, w_cat)


def _sc_body(vpos, y, out, table_sh, table_v, linb, valb,
             srcidx, idx0, idx1, px, py, pz, qx, qy, qz, tmp0, tmp1, acc0,
             gsem, wsem2, wsem):
    cid = lax.axis_index("c")
    sid = lax.axis_index("s")
    wid = cid * 16 + sid

    with jax.named_scope("p0_init"):
        neg1 = jnp.full((16,), -1, jnp.int32)

        def init_body(i, carry):
            srcidx[pl.ds(i * 16, 16)] = neg1
            return carry

        lax.fori_loop(0, INIT_W // 16, init_body, None)
        pltpu.sync_copy(srcidx.at[pl.ds(0, INIT_W)],
                        table_sh.at[pl.ds(sid * INIT_W, INIT_W)])
        plsc.subcore_barrier()

    scope1 = jax.named_scope("p1_scatter")
    scope1.__enter__()
    base = sid * PER_SC
    pltpu.sync_copy(vpos.at[pl.ds(base, PER_SC)], px)
    pltpu.sync_copy(vpos.at[pl.ds(NP + base, PER_SC)], py)
    pltpu.sync_copy(vpos.at[pl.ds(2 * NP + base, PER_SC)], pz)
    iota16 = lax.iota(jnp.int32, 16)
    for k in range(PER_SC // 128):
        for jj in range(8):
            off = k * 128 + jj * 16
            vx = px[pl.ds(off, 16)]
            vy = py[pl.ds(off, 16)]
            vz = pz[pl.ds(off, 16)]
            lin16 = (vx + 1) * (GP * GP) + (vy + 1) * GP + (vz + 1)
            linb[k, pl.ds(jj * 16, 16)] = lin16
            valb[k, pl.ds(jj * 16, 16)] = base + off + iota16
    for k in range(PER_SC // 128):
        pltpu.sync_copy(valb.at[k], table_sh.at[linb.at[k]])
    plsc.subcore_barrier()
    scope1.__exit__(None, None, None)

    scope2 = jax.named_scope("p2_nidx")
    scope2.__enter__()
    pltpu.sync_copy(table_sh, table_v)
    rbase = wid * ROWS
    lrbase = sid * ROWS
    pltpu.sync_copy(vpos.at[pl.ds(rbase, ROWS)], qx)
    pltpu.sync_copy(vpos.at[pl.ds(NP + rbase, ROWS)], qy)
    pltpu.sync_copy(vpos.at[pl.ds(2 * NP + rbase, ROWS)], qz)

    def gath_body(i, carry):
        off = i * 16
        vx = qx[pl.ds(off, 16)]
        vy = qy[pl.ds(off, 16)]
        vz = qz[pl.ds(off, 16)]
        lin16 = (vx + 1) * (GP * GP) + (vy + 1) * GP + (vz + 1)
        zspread = (rbase + off + iota16) % (NP - N)
        for o in range(NOFF):
            nidx = plsc.load_gather(table_v, [lin16 + DOFF[o]])
            srcidx[pl.ds(o * ROWS + off, 16)] = jnp.where(
                nidx >= 0, nidx * NOFF + o, INVALID + zspread * NOFF + o)
        return carry

    lax.fori_loop(0, ROWS // 16, gath_body, None)
    scope2.__exit__(None, None, None)

    scope3 = jax.named_scope("p3_accum")
    scope3.__enter__()
    def _gather(o, ch, buf, idxb, sem):
        for g in range(CH // 16):
            idxb[pl.ds(g * 16, 16)] = srcidx[
                pl.ds(o * ROWS + ch * CH + g * 16, 16)]
        return pltpu.async_copy(y.at[idxb], buf, sem)

    def _drain(buf, sem):
        pltpu.make_async_copy(y.at[pl.ds(0, CH)], buf, sem).wait()

    def _set_acc(buf):
        def body(r, carry):
            for c in range(C // 16):
                sl = pl.ds(c * 16, 16)
                acc0[r, sl] = buf[r, sl]
            return carry
        lax.fori_loop(0, CH, body, None)

    def _add_acc(buf):
        def body(r, carry):
            for c in range(C // 16):
                sl = pl.ds(c * 16, 16)
                plsc.addupdate(acc0.at[r, sl], buf[r, sl])
            return carry
        lax.fori_loop(0, CH, body, None)

    def chunk_body(ch, carry):
        @pl.when(ch > 0)
        def _():
            _drain(acc0, wsem)
        _gather(0, ch, tmp0, idx0, gsem)
        _drain(tmp0, gsem)
        _gather(1, ch, tmp1, idx1, wsem2)
        _set_acc(tmp0)
        _gather(2, ch, tmp0, idx0, gsem)

        def pair(t, c2):
            o1 = 2 * t + 1
            _drain(tmp1, wsem2)
            _add_acc(tmp1)

            @pl.when(o1 + 2 < NOFF)
            def _():
                _gather(o1 + 2, ch, tmp1, idx1, wsem2)
            _drain(tmp0, gsem)
            _add_acc(tmp0)

            @pl.when(o1 + 3 < NOFF)
            def _():
                _gather(o1 + 3, ch, tmp0, idx0, gsem)
            return c2

        lax.fori_loop(0, (NOFF - 3) // 2, pair, None)
        _drain(tmp1, wsem2)
        _add_acc(tmp1)
        _drain(tmp0, gsem)
        _add_acc(tmp0)
        pltpu.async_copy(acc0, out.at[pl.ds(rbase + ch * CH, CH)], wsem)
        return carry

    lax.fori_loop(0, NCH, chunk_body, None)
    _drain(acc0, wsem)
    scope3.__exit__(None, None, None)


def _sc_gather_add(vpos, y2d):
    mesh = plsc.VectorSubcoreMesh(core_axis_name="c", subcore_axis_name="s")
    return pl.kernel(
        _sc_body,
        out_type=jax.ShapeDtypeStruct((NP, C), jnp.float32),
        mesh=mesh,
        scratch_types=[
            pltpu.VMEM_SHARED((TAB,), jnp.int32),
            pltpu.VMEM((TAB,), jnp.int32),
            pltpu.VMEM((PER_SC // 128, 128), jnp.int32),
            pltpu.VMEM((PER_SC // 128, 128), jnp.int32),
            pltpu.VMEM((NOFF * ROWS,), jnp.int32),
            pltpu.VMEM((CH,), jnp.int32),
            pltpu.VMEM((CH,), jnp.int32),
            pltpu.VMEM((PER_SC,), jnp.int32),
            pltpu.VMEM((PER_SC,), jnp.int32),
            pltpu.VMEM((PER_SC,), jnp.int32),
            pltpu.VMEM((ROWS,), jnp.int32),
            pltpu.VMEM((ROWS,), jnp.int32),
            pltpu.VMEM((ROWS,), jnp.int32),
            pltpu.VMEM((CH, C), jnp.float32),
            pltpu.VMEM((CH, C), jnp.float32),
            pltpu.VMEM((CH, C), jnp.float32),
            pltpu.SemaphoreType.DMA,
            pltpu.SemaphoreType.DMA,
            pltpu.SemaphoreType.DMA,
        ],
        compiler_params=pltpu.CompilerParams(needs_layout_passes=False),
    )(vpos, y2d)


def kernel(features, inp_positions, W, voxel_size=1.0):
    f_pad = jnp.zeros((NP, C), jnp.float32).at[:N].set(features)
    v = jnp.floor(inp_positions / voxel_size).astype(jnp.int32)
    vpos = (jnp.full((3, NP), G + 1, jnp.int32).at[:, :N].set(v.T)
            .reshape(3 * NP))
    w_cat = W.reshape(NOFF, C, C).transpose(1, 0, 2).reshape(C, NOFF * C)

    y = _big_matmul(f_pad.astype(jnp.bfloat16), w_cat.astype(jnp.bfloat16))
    y2d = y.reshape(NP * NOFF, C)
    out = _sc_gather_add(vpos, y2d)
    return out[:N]


# speedup vs baseline: 34.8688x; 1.4782x over previous
import jax
import jax.numpy as jnp
from jax import lax
from jax.experimental import pallas as pl
from jax.experimental.pallas import tpu as pltpu
from jax.experimental.pallas import tpu_sc as plsc

N = 10000
G = 32
GP = G + 2
C = 256
NOFF = 27
NP = 10240
TAB = 41984
PER_SC = NP // 16
ROWS = NP // 32
CH = 64
NCH = ROWS // CH
INVALID = N * NOFF
INIT_W = TAB // 16
SC_ROWS = 16 * ROWS
DOFF = [dx * GP * GP + dy * GP + dz
        for dx in (-1, 0, 1) for dy in (-1, 0, 1) for dz in (-1, 0, 1)]


def _mm_body(f_ref, w_ref, y_ref):
    y_ref[0] = jnp.dot(f_ref[...], w_ref[0],
                       preferred_element_type=jnp.float32)


def _big_matmul(f_pad, w_r):
    bm = 2048
    grid = (NOFF, NP // bm)
    return pl.pallas_call(
        _mm_body,
        grid=grid,
        in_specs=[
            pl.BlockSpec((bm, C), lambda o, i: (i, 0)),
            pl.BlockSpec((1, C, C), lambda o, i: (o, 0, 0)),
        ],
        out_specs=pl.BlockSpec((1, bm, C), lambda o, i: (o, i, 0)),
        out_shape=jax.ShapeDtypeStruct((NOFF, NP, C), jnp.float32),
    )(f_pad, w_r)


def _sc_body(vpos, y, out, table_sh, table_v, linb, valb,
             srcidx, idx0, idx1, px, py, pz, qx, qy, qz, tmp0, tmp1, acc0,
             gsem, wsem2, wsem):
    cid = lax.axis_index("c")
    sid = lax.axis_index("s")
    wid = cid * 16 + sid

    with jax.named_scope("p0_init"):
        neg1 = jnp.full((16,), -1, jnp.int32)

        def init_body(i, carry):
            srcidx[pl.ds(i * 16, 16)] = neg1
            return carry

        lax.fori_loop(0, INIT_W // 16, init_body, None)
        pltpu.sync_copy(srcidx.at[pl.ds(0, INIT_W)],
                        table_sh.at[pl.ds(sid * INIT_W, INIT_W)])
        plsc.subcore_barrier()

    scope1 = jax.named_scope("p1_scatter")
    scope1.__enter__()
    base = sid * PER_SC
    pltpu.sync_copy(vpos.at[pl.ds(base, PER_SC)], px)
    pltpu.sync_copy(vpos.at[pl.ds(NP + base, PER_SC)], py)
    pltpu.sync_copy(vpos.at[pl.ds(2 * NP + base, PER_SC)], pz)
    iota16 = lax.iota(jnp.int32, 16)
    for k in range(PER_SC // 128):
        for jj in range(8):
            off = k * 128 + jj * 16
            vx = px[pl.ds(off, 16)]
            vy = py[pl.ds(off, 16)]
            vz = pz[pl.ds(off, 16)]
            lin16 = (vx + 1) * (GP * GP) + (vy + 1) * GP + (vz + 1)
            linb[k, pl.ds(jj * 16, 16)] = lin16
            valb[k, pl.ds(jj * 16, 16)] = base + off + iota16
    for k in range(PER_SC // 128):
        pltpu.sync_copy(valb.at[k], table_sh.at[linb.at[k]])
    plsc.subcore_barrier()
    scope1.__exit__(None, None, None)

    scope2 = jax.named_scope("p2_nidx")
    scope2.__enter__()
    pltpu.sync_copy(table_sh, table_v)
    rbase = wid * ROWS
    lrbase = sid * ROWS
    pltpu.sync_copy(vpos.at[pl.ds(rbase, ROWS)], qx)
    pltpu.sync_copy(vpos.at[pl.ds(NP + rbase, ROWS)], qy)
    pltpu.sync_copy(vpos.at[pl.ds(2 * NP + rbase, ROWS)], qz)

    def gath_body(i, carry):
        off = i * 16
        vx = qx[pl.ds(off, 16)]
        vy = qy[pl.ds(off, 16)]
        vz = qz[pl.ds(off, 16)]
        lin16 = (vx + 1) * (GP * GP) + (vy + 1) * GP + (vz + 1)
        zspread = (rbase + off + iota16) % (NP - N)
        for o in range(NOFF):
            nidx = plsc.load_gather(table_v, [lin16 + DOFF[o]])
            srcidx[pl.ds(o * ROWS + off, 16)] = jnp.where(
                nidx >= 0, o * NP + nidx, o * NP + N + zspread)
        return carry

    lax.fori_loop(0, ROWS // 16, gath_body, None)
    scope2.__exit__(None, None, None)

    scope3 = jax.named_scope("p3_accum")
    scope3.__enter__()
    def _gather(o, ch, buf, idxb, sem):
        for g in range(CH // 16):
            idxb[pl.ds(g * 16, 16)] = srcidx[
                pl.ds(o * ROWS + ch * CH + g * 16, 16)]
        return pltpu.async_copy(y.at[idxb], buf, sem)

    def _drain(buf, sem):
        pltpu.make_async_copy(y.at[pl.ds(0, CH)], buf, sem).wait()

    def _set_acc(buf):
        def body(r, carry):
            for c in range(C // 16):
                sl = pl.ds(c * 16, 16)
                acc0[r, sl] = buf[r, sl]
            return carry
        lax.fori_loop(0, CH, body, None)

    def _add_acc(buf):
        def body(r, carry):
            for c in range(C // 16):
                sl = pl.ds(c * 16, 16)
                plsc.addupdate(acc0.at[r, sl], buf[r, sl])
            return carry
        lax.fori_loop(0, CH, body, None)

    def chunk_body(ch, carry):
        @pl.when(ch > 0)
        def _():
            _drain(acc0, wsem)
        _gather(0, ch, tmp0, idx0, gsem)
        _drain(tmp0, gsem)
        _gather(1, ch, tmp1, idx1, wsem2)
        _set_acc(tmp0)
        _gather(2, ch, tmp0, idx0, gsem)

        def pair(t, c2):
            o1 = 2 * t + 1
            _drain(tmp1, wsem2)
            _add_acc(tmp1)

            @pl.when(o1 + 2 < NOFF)
            def _():
                _gather(o1 + 2, ch, tmp1, idx1, wsem2)
            _drain(tmp0, gsem)
            _add_acc(tmp0)

            @pl.when(o1 + 3 < NOFF)
            def _():
                _gather(o1 + 3, ch, tmp0, idx0, gsem)
            return c2

        lax.fori_loop(0, (NOFF - 3) // 2, pair, None)
        _drain(tmp1, wsem2)
        _add_acc(tmp1)
        _drain(tmp0, gsem)
        _add_acc(tmp0)
        pltpu.async_copy(acc0, out.at[pl.ds(rbase + ch * CH, CH)], wsem)
        return carry

    lax.fori_loop(0, NCH, chunk_body, None)
    _drain(acc0, wsem)
    scope3.__exit__(None, None, None)


def _sc_gather_add(vpos, y2d):
    mesh = plsc.VectorSubcoreMesh(core_axis_name="c", subcore_axis_name="s")
    return pl.kernel(
        _sc_body,
        out_type=jax.ShapeDtypeStruct((NP, C), jnp.float32),
        mesh=mesh,
        scratch_types=[
            pltpu.VMEM_SHARED((TAB,), jnp.int32),
            pltpu.VMEM((TAB,), jnp.int32),
            pltpu.VMEM((PER_SC // 128, 128), jnp.int32),
            pltpu.VMEM((PER_SC // 128, 128), jnp.int32),
            pltpu.VMEM((NOFF * ROWS,), jnp.int32),
            pltpu.VMEM((CH,), jnp.int32),
            pltpu.VMEM((CH,), jnp.int32),
            pltpu.VMEM((PER_SC,), jnp.int32),
            pltpu.VMEM((PER_SC,), jnp.int32),
            pltpu.VMEM((PER_SC,), jnp.int32),
            pltpu.VMEM((ROWS,), jnp.int32),
            pltpu.VMEM((ROWS,), jnp.int32),
            pltpu.VMEM((ROWS,), jnp.int32),
            pltpu.VMEM((CH, C), jnp.float32),
            pltpu.VMEM((CH, C), jnp.float32),
            pltpu.VMEM((CH, C), jnp.float32),
            pltpu.SemaphoreType.DMA,
            pltpu.SemaphoreType.DMA,
            pltpu.SemaphoreType.DMA,
        ],
        compiler_params=pltpu.CompilerParams(needs_layout_passes=False),
    )(vpos, y2d)


def kernel(features, inp_positions, W, voxel_size=1.0):
    f_pad = jnp.zeros((NP, C), jnp.float32).at[:N].set(features)
    v = jnp.floor(inp_positions / voxel_size).astype(jnp.int32)
    vpos = (jnp.full((3, NP), G + 1, jnp.int32).at[:, :N].set(v.T)
            .reshape(3 * NP))
    w_r = W.reshape(NOFF, C, C)

    y = _big_matmul(f_pad.astype(jnp.bfloat16), w_r.astype(jnp.bfloat16))
    y2d = y.reshape(NOFF * NP, C)
    out = _sc_gather_add(vpos, y2d)
    return out[:N]


# speedup vs baseline: 34.9240x; 1.0016x over previous
import jax
import jax.numpy as jnp
from jax import lax
from jax.experimental import pallas as pl
from jax.experimental.pallas import tpu as pltpu
from jax.experimental.pallas import tpu_sc as plsc

N = 10000
G = 32
GP = G + 2
C = 256
NOFF = 27
NP = 10240
TAB = 41984
PER_SC = NP // 16
ROWS = NP // 32
CH = 64
NCH = ROWS // CH
INVALID = N * NOFF
INIT_W = TAB // 16
SC_ROWS = 16 * ROWS
DOFF = [dx * GP * GP + dy * GP + dz
        for dx in (-1, 0, 1) for dy in (-1, 0, 1) for dz in (-1, 0, 1)]


def _mm_body(f_ref, w_ref, y_ref):
    y_ref[0] = jnp.dot(f_ref[...], w_ref[0],
                       preferred_element_type=jnp.float32)


def _big_matmul(f_pad, w_r):
    bm = 2048
    grid = (NOFF, NP // bm)
    return pl.pallas_call(
        _mm_body,
        grid=grid,
        in_specs=[
            pl.BlockSpec((bm, C), lambda o, i: (i, 0)),
            pl.BlockSpec((1, C, C), lambda o, i: (o, 0, 0)),
        ],
        out_specs=pl.BlockSpec((1, bm, C), lambda o, i: (o, i, 0)),
        out_shape=jax.ShapeDtypeStruct((NOFF, NP, C), jnp.float32),
    )(f_pad, w_r)


def _sc_body(vpos, y, out, table_sh, table_v, linb, valb,
             srcidx, idx0, idx1, px, py, pz, qx, qy, qz, tmp0, tmp1, acc0,
             gsem, wsem2, wsem):
    cid = lax.axis_index("c")
    sid = lax.axis_index("s")
    wid = cid * 16 + sid

    with jax.named_scope("p0_init"):
        neg1 = jnp.full((16,), -1, jnp.int32)

        def init_body(i, carry):
            srcidx[pl.ds(i * 16, 16)] = neg1
            return carry

        lax.fori_loop(0, INIT_W // 16, init_body, None)
        pltpu.sync_copy(srcidx.at[pl.ds(0, INIT_W)],
                        table_sh.at[pl.ds(sid * INIT_W, INIT_W)])
        plsc.subcore_barrier()

    scope1 = jax.named_scope("p1_scatter")
    scope1.__enter__()
    base = sid * PER_SC
    pltpu.sync_copy(vpos.at[pl.ds(base, PER_SC)], px)
    pltpu.sync_copy(vpos.at[pl.ds(NP + base, PER_SC)], py)
    pltpu.sync_copy(vpos.at[pl.ds(2 * NP + base, PER_SC)], pz)
    iota16 = lax.iota(jnp.int32, 16)
    for k in range(PER_SC // 128):
        for jj in range(8):
            off = k * 128 + jj * 16
            vx = px[pl.ds(off, 16)]
            vy = py[pl.ds(off, 16)]
            vz = pz[pl.ds(off, 16)]
            lin16 = (vx + 1) * (GP * GP) + (vy + 1) * GP + (vz + 1)
            linb[k, pl.ds(jj * 16, 16)] = lin16
            valb[k, pl.ds(jj * 16, 16)] = base + off + iota16
    for k in range(PER_SC // 128):
        pltpu.sync_copy(valb.at[k], table_sh.at[linb.at[k]])
    plsc.subcore_barrier()
    scope1.__exit__(None, None, None)

    scope2 = jax.named_scope("p2_nidx")
    scope2.__enter__()
    pltpu.sync_copy(table_sh, table_v)
    rbase = wid * ROWS
    lrbase = sid * ROWS
    pltpu.sync_copy(vpos.at[pl.ds(rbase, ROWS)], qx)
    pltpu.sync_copy(vpos.at[pl.ds(NP + rbase, ROWS)], qy)
    pltpu.sync_copy(vpos.at[pl.ds(2 * NP + rbase, ROWS)], qz)

    def gath_body(i, carry):
        off = i * 16
        vx = qx[pl.ds(off, 16)]
        vy = qy[pl.ds(off, 16)]
        vz = qz[pl.ds(off, 16)]
        lin16 = (vx + 1) * (GP * GP) + (vy + 1) * GP + (vz + 1)
        zspread = (rbase + off + iota16) % (NP - N)
        for o in range(NOFF):
            nidx = plsc.load_gather(table_v, [lin16 + DOFF[o]])
            srcidx[pl.ds(o * ROWS + off, 16)] = jnp.where(
                nidx >= 0, o * NP + nidx, o * NP + N + zspread)
        return carry

    lax.fori_loop(0, ROWS // 16, gath_body, None)
    scope2.__exit__(None, None, None)

    scope3 = jax.named_scope("p3_accum")
    scope3.__enter__()
    def _gather(o, ch, buf, idxb, sem):
        for g in range(CH // 16):
            idxb[pl.ds(g * 16, 16)] = srcidx[
                pl.ds(o * ROWS + ch * CH + g * 16, 16)]
        return pltpu.async_copy(y.at[idxb], buf, sem)

    def _drain(buf, sem):
        pltpu.make_async_copy(y.at[pl.ds(0, CH)], buf, sem).wait()

    def _set_acc(buf):
        @plsc.parallel_loop(0, CH, 1)
        def body(r):
            for c in range(C // 16):
                sl = pl.ds(c * 16, 16)
                acc0[r, sl] = buf[r, sl]

    def _add_acc(buf):
        @plsc.parallel_loop(0, CH, 1)
        def body(r):
            for c in range(C // 16):
                sl = pl.ds(c * 16, 16)
                plsc.addupdate(acc0.at[r, sl], buf[r, sl])

    def chunk_body(ch, carry):
        @pl.when(ch > 0)
        def _():
            _drain(acc0, wsem)
        _gather(0, ch, tmp0, idx0, gsem)
        _drain(tmp0, gsem)
        _gather(1, ch, tmp1, idx1, wsem2)
        _set_acc(tmp0)
        _gather(2, ch, tmp0, idx0, gsem)

        def pair(t, c2):
            o1 = 2 * t + 1
            _drain(tmp1, wsem2)
            _add_acc(tmp1)

            @pl.when(o1 + 2 < NOFF)
            def _():
                _gather(o1 + 2, ch, tmp1, idx1, wsem2)
            _drain(tmp0, gsem)
            _add_acc(tmp0)

            @pl.when(o1 + 3 < NOFF)
            def _():
                _gather(o1 + 3, ch, tmp0, idx0, gsem)
            return c2

        lax.fori_loop(0, (NOFF - 3) // 2, pair, None)
        _drain(tmp1, wsem2)
        _add_acc(tmp1)
        _drain(tmp0, gsem)
        _add_acc(tmp0)
        pltpu.async_copy(acc0, out.at[pl.ds(rbase + ch * CH, CH)], wsem)
        return carry

    lax.fori_loop(0, NCH, chunk_body, None)
    _drain(acc0, wsem)
    scope3.__exit__(None, None, None)


def _sc_gather_add(vpos, y2d):
    mesh = plsc.VectorSubcoreMesh(core_axis_name="c", subcore_axis_name="s")
    return pl.kernel(
        _sc_body,
        out_type=jax.ShapeDtypeStruct((NP, C), jnp.float32),
        mesh=mesh,
        scratch_types=[
            pltpu.VMEM_SHARED((TAB,), jnp.int32),
            pltpu.VMEM((TAB,), jnp.int32),
            pltpu.VMEM((PER_SC // 128, 128), jnp.int32),
            pltpu.VMEM((PER_SC // 128, 128), jnp.int32),
            pltpu.VMEM((NOFF * ROWS,), jnp.int32),
            pltpu.VMEM((CH,), jnp.int32),
            pltpu.VMEM((CH,), jnp.int32),
            pltpu.VMEM((PER_SC,), jnp.int32),
            pltpu.VMEM((PER_SC,), jnp.int32),
            pltpu.VMEM((PER_SC,), jnp.int32),
            pltpu.VMEM((ROWS,), jnp.int32),
            pltpu.VMEM((ROWS,), jnp.int32),
            pltpu.VMEM((ROWS,), jnp.int32),
            pltpu.VMEM((CH, C), jnp.float32),
            pltpu.VMEM((CH, C), jnp.float32),
            pltpu.VMEM((CH, C), jnp.float32),
            pltpu.SemaphoreType.DMA,
            pltpu.SemaphoreType.DMA,
            pltpu.SemaphoreType.DMA,
        ],
        compiler_params=pltpu.CompilerParams(needs_layout_passes=False),
    )(vpos, y2d)


def kernel(features, inp_positions, W, voxel_size=1.0):
    f_pad = jnp.zeros((NP, C), jnp.float32).at[:N].set(features)
    v = jnp.floor(inp_positions / voxel_size).astype(jnp.int32)
    vpos = (jnp.full((3, NP), G + 1, jnp.int32).at[:, :N].set(v.T)
            .reshape(3 * NP))
    w_r = W.reshape(NOFF, C, C)

    y = _big_matmul(f_pad.astype(jnp.bfloat16), w_r.astype(jnp.bfloat16))
    y2d = y.reshape(NOFF * NP, C)
    out = _sc_gather_add(vpos, y2d)
    return out[:N]


# speedup vs baseline: 44.7499x; 1.2813x over previous
import jax
import jax.numpy as jnp
from jax import lax
from jax.experimental import pallas as pl
from jax.experimental.pallas import tpu as pltpu
from jax.experimental.pallas import tpu_sc as plsc

N = 10000
G = 32
GP = G + 2
C = 256
NOFF = 27
NP = 10240
TAB = 41984
PER_SC = NP // 16
ROWS = NP // 32
CH = 64
NCH = ROWS // CH
INVALID = N * NOFF
INIT_W = TAB // 16
SC_ROWS = 16 * ROWS
DOFF = [dx * GP * GP + dy * GP + dz
        for dx in (-1, 0, 1) for dy in (-1, 0, 1) for dz in (-1, 0, 1)]


def _mm_body(f_ref, w_ref, y_ref):
    y_ref[0] = jnp.dot(f_ref[...], w_ref[0],
                       preferred_element_type=jnp.float32)


def _big_matmul(f_pad, w_r):
    grid = (NOFF,)
    return pl.pallas_call(
        _mm_body,
        grid=grid,
        in_specs=[
            pl.BlockSpec((NP, C), lambda o: (0, 0)),
            pl.BlockSpec((1, C, C), lambda o: (o, 0, 0)),
        ],
        out_specs=pl.BlockSpec((1, NP, C), lambda o: (o, 0, 0)),
        out_shape=jax.ShapeDtypeStruct((NOFF, NP, C), jnp.float32),
    )(f_pad, w_r)


def _sc_body(vpos, y, out, table_sh, table_v, linb, valb,
             srcidx, idx0, idx1, px, py, pz, qx, qy, qz, tmp0, tmp1, acc0,
             gsem, wsem2, wsem):
    cid = lax.axis_index("c")
    sid = lax.axis_index("s")
    wid = cid * 16 + sid

    with jax.named_scope("p0_init"):
        neg1 = jnp.full((16,), -1, jnp.int32)

        def init_body(i, carry):
            srcidx[pl.ds(i * 16, 16)] = neg1
            return carry

        lax.fori_loop(0, INIT_W // 16, init_body, None)
        pltpu.sync_copy(srcidx.at[pl.ds(0, INIT_W)],
                        table_sh.at[pl.ds(sid * INIT_W, INIT_W)])
        plsc.subcore_barrier()

    scope1 = jax.named_scope("p1_scatter")
    scope1.__enter__()
    base = sid * PER_SC
    pltpu.sync_copy(vpos.at[pl.ds(base, PER_SC)], px)
    pltpu.sync_copy(vpos.at[pl.ds(NP + base, PER_SC)], py)
    pltpu.sync_copy(vpos.at[pl.ds(2 * NP + base, PER_SC)], pz)
    iota16 = lax.iota(jnp.int32, 16)
    for k in range(PER_SC // 128):
        for jj in range(8):
            off = k * 128 + jj * 16
            vx = px[pl.ds(off, 16)]
            vy = py[pl.ds(off, 16)]
            vz = pz[pl.ds(off, 16)]
            lin16 = (vx + 1) * (GP * GP) + (vy + 1) * GP + (vz + 1)
            linb[k, pl.ds(jj * 16, 16)] = lin16
            valb[k, pl.ds(jj * 16, 16)] = base + off + iota16
    for k in range(PER_SC // 128):
        pltpu.sync_copy(valb.at[k], table_sh.at[linb.at[k]])
    plsc.subcore_barrier()
    scope1.__exit__(None, None, None)

    scope2 = jax.named_scope("p2_nidx")
    scope2.__enter__()
    pltpu.sync_copy(table_sh, table_v)
    rbase = wid * ROWS
    lrbase = sid * ROWS
    pltpu.sync_copy(vpos.at[pl.ds(rbase, ROWS)], qx)
    pltpu.sync_copy(vpos.at[pl.ds(NP + rbase, ROWS)], qy)
    pltpu.sync_copy(vpos.at[pl.ds(2 * NP + rbase, ROWS)], qz)

    def gath_body(i, carry):
        off = i * 16
        vx = qx[pl.ds(off, 16)]
        vy = qy[pl.ds(off, 16)]
        vz = qz[pl.ds(off, 16)]
        lin16 = (vx + 1) * (GP * GP) + (vy + 1) * GP + (vz + 1)
        zspread = (rbase + off + iota16) % (NP - N)
        for o in range(NOFF):
            nidx = plsc.load_gather(table_v, [lin16 + DOFF[o]])
            srcidx[pl.ds(o * ROWS + off, 16)] = jnp.where(
                nidx >= 0, o * NP + nidx, o * NP + N + zspread)
        return carry

    lax.fori_loop(0, ROWS // 16, gath_body, None)
    scope2.__exit__(None, None, None)

    scope3 = jax.named_scope("p3_accum")
    scope3.__enter__()
    def _gather(o, ch, buf, idxb, sem):
        for g in range(CH // 16):
            idxb[pl.ds(g * 16, 16)] = srcidx[
                pl.ds(o * ROWS + ch * CH + g * 16, 16)]
        return pltpu.async_copy(y.at[idxb], buf, sem)

    def _drain(buf, sem):
        pltpu.make_async_copy(y.at[pl.ds(0, CH)], buf, sem).wait()

    def _set_acc(buf):
        @plsc.parallel_loop(0, CH, 1)
        def body(r):
            for c in range(C // 16):
                sl = pl.ds(c * 16, 16)
                acc0[r, sl] = buf[r, sl]

    def _add_acc(buf):
        @plsc.parallel_loop(0, CH, 1)
        def body(r):
            for c in range(C // 16):
                sl = pl.ds(c * 16, 16)
                plsc.addupdate(acc0.at[r, sl], buf[r, sl])

    def chunk_body(ch, carry):
        @pl.when(ch > 0)
        def _():
            _drain(acc0, wsem)
        _gather(0, ch, tmp0, idx0, gsem)
        _drain(tmp0, gsem)
        _gather(1, ch, tmp1, idx1, wsem2)
        _set_acc(tmp0)
        _gather(2, ch, tmp0, idx0, gsem)

        def pair(t, c2):
            o1 = 2 * t + 1
            _drain(tmp1, wsem2)
            _add_acc(tmp1)

            @pl.when(o1 + 2 < NOFF)
            def _():
                _gather(o1 + 2, ch, tmp1, idx1, wsem2)
            _drain(tmp0, gsem)
            _add_acc(tmp0)

            @pl.when(o1 + 3 < NOFF)
            def _():
                _gather(o1 + 3, ch, tmp0, idx0, gsem)
            return c2

        lax.fori_loop(0, (NOFF - 3) // 2, pair, None)
        _drain(tmp1, wsem2)
        _add_acc(tmp1)
        _drain(tmp0, gsem)
        _add_acc(tmp0)
        pltpu.async_copy(acc0, out.at[pl.ds(rbase + ch * CH, CH)], wsem)
        return carry

    lax.fori_loop(0, NCH, chunk_body, None)
    _drain(acc0, wsem)
    scope3.__exit__(None, None, None)


def _sc_gather_add(vpos, y2d):
    mesh = plsc.VectorSubcoreMesh(core_axis_name="c", subcore_axis_name="s")
    return pl.kernel(
        _sc_body,
        out_type=jax.ShapeDtypeStruct((NP, C), jnp.float32),
        mesh=mesh,
        scratch_types=[
            pltpu.VMEM_SHARED((TAB,), jnp.int32),
            pltpu.VMEM((TAB,), jnp.int32),
            pltpu.VMEM((PER_SC // 128, 128), jnp.int32),
            pltpu.VMEM((PER_SC // 128, 128), jnp.int32),
            pltpu.VMEM((NOFF * ROWS,), jnp.int32),
            pltpu.VMEM((CH,), jnp.int32),
            pltpu.VMEM((CH,), jnp.int32),
            pltpu.VMEM((PER_SC,), jnp.int32),
            pltpu.VMEM((PER_SC,), jnp.int32),
            pltpu.VMEM((PER_SC,), jnp.int32),
            pltpu.VMEM((ROWS,), jnp.int32),
            pltpu.VMEM((ROWS,), jnp.int32),
            pltpu.VMEM((ROWS,), jnp.int32),
            pltpu.VMEM((CH, C), jnp.float32),
            pltpu.VMEM((CH, C), jnp.float32),
            pltpu.VMEM((CH, C), jnp.float32),
            pltpu.SemaphoreType.DMA,
            pltpu.SemaphoreType.DMA,
            pltpu.SemaphoreType.DMA,
        ],
        compiler_params=pltpu.CompilerParams(needs_layout_passes=False),
    )(vpos, y2d)


def kernel(features, inp_positions, W, voxel_size=1.0):
    f_pad = jnp.zeros((NP, C), jnp.float32).at[:N].set(features)
    v = jnp.floor(inp_positions / voxel_size).astype(jnp.int32)
    vpos = (jnp.full((3, NP), G + 1, jnp.int32).at[:, :N].set(v.T)
            .reshape(3 * NP))
    w_r = W.reshape(NOFF, C, C)

    y = _big_matmul(f_pad.astype(jnp.bfloat16), w_r.astype(jnp.bfloat16))
    y2d = y.reshape(NOFF * NP, C)
    out = _sc_gather_add(vpos, y2d)
    return out[:N]


# speedup vs baseline: 44.8210x; 1.0016x over previous
import jax
import jax.numpy as jnp
from jax import lax
from jax.experimental import pallas as pl
from jax.experimental.pallas import tpu as pltpu
from jax.experimental.pallas import tpu_sc as plsc

N = 10000
G = 32
GP = G + 2
C = 256
NOFF = 27
NP = 10240
TAB = 41984
PER_SC = NP // 16
ROWS = NP // 32
CH = 64
NCH = ROWS // CH
INVALID = N * NOFF
INIT_W = TAB // 16
SC_ROWS = 16 * ROWS
DOFF = [dx * GP * GP + dy * GP + dz
        for dx in (-1, 0, 1) for dy in (-1, 0, 1) for dz in (-1, 0, 1)]


def _mm_body(f_ref, w_ref, y_ref):
    y_ref[0] = jnp.dot(f_ref[...], w_ref[0],
                       preferred_element_type=jnp.float32)


def _big_matmul(f_pad, w_r):
    grid = (NOFF,)
    return pl.pallas_call(
        _mm_body,
        grid=grid,
        in_specs=[
            pl.BlockSpec((NP, C), lambda o: (0, 0)),
            pl.BlockSpec((1, C, C), lambda o: (o, 0, 0)),
        ],
        out_specs=pl.BlockSpec((1, NP, C), lambda o: (o, 0, 0)),
        out_shape=jax.ShapeDtypeStruct((NOFF, NP, C), jnp.float32),
    )(f_pad, w_r)


def _sc_body(vpos, y, out, table_sh, table_v, linb, valb,
             srcidx, idx0, idx1, px, py, pz, qx, qy, qz, tmp0, tmp1, acc0,
             gsem, wsem2, wsem):
    cid = lax.axis_index("c")
    sid = lax.axis_index("s")
    wid = cid * 16 + sid

    with jax.named_scope("p0_init"):
        neg1 = jnp.full((16,), -1, jnp.int32)

        def init_body(i, carry):
            srcidx[pl.ds(i * 16, 16)] = neg1
            return carry

        lax.fori_loop(0, INIT_W // 16, init_body, None)
        pltpu.sync_copy(srcidx.at[pl.ds(0, INIT_W)],
                        table_sh.at[pl.ds(sid * INIT_W, INIT_W)])
        plsc.subcore_barrier()

    scope1 = jax.named_scope("p1_scatter")
    scope1.__enter__()
    base = sid * PER_SC
    pltpu.sync_copy(vpos.at[pl.ds(base, PER_SC)], px)
    pltpu.sync_copy(vpos.at[pl.ds(NP + base, PER_SC)], py)
    pltpu.sync_copy(vpos.at[pl.ds(2 * NP + base, PER_SC)], pz)
    iota16 = lax.iota(jnp.int32, 16)
    for k in range(PER_SC // 128):
        for jj in range(8):
            off = k * 128 + jj * 16
            vx = px[pl.ds(off, 16)]
            vy = py[pl.ds(off, 16)]
            vz = pz[pl.ds(off, 16)]
            lin16 = (vx + 1) * (GP * GP) + (vy + 1) * GP + (vz + 1)
            linb[k, pl.ds(jj * 16, 16)] = lin16
            valb[k, pl.ds(jj * 16, 16)] = base + off + iota16
    for k in range(PER_SC // 128):
        pltpu.sync_copy(valb.at[k], table_sh.at[linb.at[k]])
    plsc.subcore_barrier()
    scope1.__exit__(None, None, None)

    scope2 = jax.named_scope("p2_nidx")
    scope2.__enter__()
    pltpu.sync_copy(table_sh, table_v)
    rbase = wid * ROWS
    lrbase = sid * ROWS
    pltpu.sync_copy(vpos.at[pl.ds(rbase, ROWS)], qx)
    pltpu.sync_copy(vpos.at[pl.ds(NP + rbase, ROWS)], qy)
    pltpu.sync_copy(vpos.at[pl.ds(2 * NP + rbase, ROWS)], qz)

    def gath_body(i, carry):
        off = i * 16
        vx = qx[pl.ds(off, 16)]
        vy = qy[pl.ds(off, 16)]
        vz = qz[pl.ds(off, 16)]
        lin16 = (vx + 1) * (GP * GP) + (vy + 1) * GP + (vz + 1)
        zspread = (rbase + off + iota16) % (NP - N)
        for o in range(NOFF):
            nidx = plsc.load_gather(table_v, [lin16 + DOFF[o]])
            srcidx[pl.ds(o * ROWS + off, 16)] = jnp.where(
                nidx >= 0, o * NP + nidx, o * NP + N + zspread)
        return carry

    lax.fori_loop(0, ROWS // 16, gath_body, None)
    scope2.__exit__(None, None, None)

    scope3 = jax.named_scope("p3_accum")
    scope3.__enter__()
    def _gather(o, ch, buf, idxb, sem):
        for g in range(CH // 16):
            idxb[pl.ds(g * 16, 16)] = srcidx[
                pl.ds(o * ROWS + ch * CH + g * 16, 16)]
        return pltpu.async_copy(y.at[idxb], buf, sem)

    def _drain(buf, sem):
        pltpu.make_async_copy(y.at[pl.ds(0, CH)], buf, sem).wait()

    def _set_acc(buf):
        @plsc.parallel_loop(0, CH, 1)
        def body(r):
            for c in range(C // 16):
                sl = pl.ds(c * 16, 16)
                acc0[r, sl] = buf[r, sl]

    def _add_acc(buf):
        @plsc.parallel_loop(0, CH, 1, unroll=2)
        def body(r):
            for c in range(C // 16):
                sl = pl.ds(c * 16, 16)
                plsc.addupdate(acc0.at[r, sl], buf[r, sl])

    def chunk_body(ch, carry):
        @pl.when(ch > 0)
        def _():
            _drain(acc0, wsem)
        _gather(0, ch, tmp0, idx0, gsem)
        _drain(tmp0, gsem)
        _gather(1, ch, tmp1, idx1, wsem2)
        _set_acc(tmp0)
        _gather(2, ch, tmp0, idx0, gsem)

        def pair(t, c2):
            o1 = 2 * t + 1
            _drain(tmp1, wsem2)
            _add_acc(tmp1)

            @pl.when(o1 + 2 < NOFF)
            def _():
                _gather(o1 + 2, ch, tmp1, idx1, wsem2)
            _drain(tmp0, gsem)
            _add_acc(tmp0)

            @pl.when(o1 + 3 < NOFF)
            def _():
                _gather(o1 + 3, ch, tmp0, idx0, gsem)
            return c2

        lax.fori_loop(0, (NOFF - 3) // 2, pair, None)
        _drain(tmp1, wsem2)
        _add_acc(tmp1)
        _drain(tmp0, gsem)
        _add_acc(tmp0)
        pltpu.async_copy(acc0, out.at[pl.ds(rbase + ch * CH, CH)], wsem)
        return carry

    lax.fori_loop(0, NCH, chunk_body, None)
    _drain(acc0, wsem)
    scope3.__exit__(None, None, None)


def _sc_gather_add(vpos, y2d):
    mesh = plsc.VectorSubcoreMesh(core_axis_name="c", subcore_axis_name="s")
    return pl.kernel(
        _sc_body,
        out_type=jax.ShapeDtypeStruct((NP, C), jnp.float32),
        mesh=mesh,
        scratch_types=[
            pltpu.VMEM_SHARED((TAB,), jnp.int32),
            pltpu.VMEM((TAB,), jnp.int32),
            pltpu.VMEM((PER_SC // 128, 128), jnp.int32),
            pltpu.VMEM((PER_SC // 128, 128), jnp.int32),
            pltpu.VMEM((NOFF * ROWS,), jnp.int32),
            pltpu.VMEM((CH,), jnp.int32),
            pltpu.VMEM((CH,), jnp.int32),
            pltpu.VMEM((PER_SC,), jnp.int32),
            pltpu.VMEM((PER_SC,), jnp.int32),
            pltpu.VMEM((PER_SC,), jnp.int32),
            pltpu.VMEM((ROWS,), jnp.int32),
            pltpu.VMEM((ROWS,), jnp.int32),
            pltpu.VMEM((ROWS,), jnp.int32),
            pltpu.VMEM((CH, C), jnp.float32),
            pltpu.VMEM((CH, C), jnp.float32),
            pltpu.VMEM((CH, C), jnp.float32),
            pltpu.SemaphoreType.DMA,
            pltpu.SemaphoreType.DMA,
            pltpu.SemaphoreType.DMA,
        ],
        compiler_params=pltpu.CompilerParams(needs_layout_passes=False),
    )(vpos, y2d)


def kernel(features, inp_positions, W, voxel_size=1.0):
    f_pad = jnp.zeros((NP, C), jnp.float32).at[:N].set(features)
    v = jnp.floor(inp_positions / voxel_size).astype(jnp.int32)
    vpos = (jnp.full((3, NP), G + 1, jnp.int32).at[:, :N].set(v.T)
            .reshape(3 * NP))
    w_r = W.reshape(NOFF, C, C)

    y = _big_matmul(f_pad.astype(jnp.bfloat16), w_r.astype(jnp.bfloat16))
    y2d = y.reshape(NOFF * NP, C)
    out = _sc_gather_add(vpos, y2d)
    return out[:N]
